# Initial kernel scaffold; baseline (speedup 1.0000x reference)
import jax
import jax.numpy as jnp
from jax.experimental import pallas as pl


def kernel(x):
    raise NotImplementedError("write your pallas kernel here")


# speedup vs baseline: 4.6035x; 4.6035x over previous
import jax
import jax.numpy as jnp
from jax import lax
from jax.experimental import pallas as pl
from jax.experimental.pallas import tpu as pltpu

K = 64
ROWS_PER_BLOCK = 8
N = 32768
INT_MIN = -2147483648


def _block_kernel(x_ref, o_ref, m_ref):
    x = x_ref[...]
    xi = lax.bitcast_convert_type(x, jnp.int32)
    s = jnp.where(xi >= 0, xi, jnp.bitwise_xor(xi, jnp.int32(0x7FFFFFFF)))

    def search_body(i, p):
        bit = jnp.left_shift(jnp.int32(1), jnp.int32(31) - i)
        cand = p + bit
        cnt = jnp.sum((s >= cand).astype(jnp.int32), axis=1, keepdims=True)
        return jnp.where(cnt >= K, cand, p)

    p0 = jnp.full((x.shape[0], 1), INT_MIN, dtype=jnp.int32)
    t = lax.fori_loop(0, 32, search_body, p0)

    gt = s > t
    eq = s == t
    cnt_gt = jnp.sum(gt.astype(jnp.int32), axis=1, keepdims=True)
    cnt_eq = jnp.sum(eq.astype(jnp.int32), axis=1, keepdims=True)
    need = K - cnt_gt

    m_ref[...] = jnp.full((x.shape[0], 1), jnp.int32(N - 1))

    @pl.when(jnp.any(cnt_eq > need))
    def _tie_search():
        iota = lax.broadcasted_iota(jnp.int32, x.shape, 1)
        eqi = eq.astype(jnp.int32)

        def tie_body(i, pm):
            bit = jnp.left_shift(jnp.int32(1), jnp.int32(14) - i)
            cand = pm & ~bit
            f = jnp.sum(jnp.where(iota <= cand, eqi, 0),
                        axis=1, keepdims=True)
            return jnp.where(f >= need, cand, pm)

        pm0 = jnp.full((x.shape[0], 1), jnp.int32(N - 1))
        m_ref[...] = lax.fori_loop(0, 15, tie_body, pm0)

    m = m_ref[...]
    iota = lax.broadcasted_iota(jnp.int32, x.shape, 1)
    keep = gt | (eq & (iota <= m))
    o_ref[...] = jnp.where(keep, x, -jnp.inf)


def kernel(x):
    rows, n = x.shape
    grid = (rows // ROWS_PER_BLOCK,)
    return pl.pallas_call(
        _block_kernel,
        grid=grid,
        in_specs=[pl.BlockSpec((ROWS_PER_BLOCK, n), lambda i: (i, 0))],
        out_specs=pl.BlockSpec((ROWS_PER_BLOCK, n), lambda i: (i, 0)),
        out_shape=jax.ShapeDtypeStruct((rows, n), x.dtype),
        scratch_shapes=[pltpu.VMEM((ROWS_PER_BLOCK, 1), jnp.int32)],
    )(x)
